# Initial kernel scaffold; baseline (speedup 1.0000x reference)
import jax
import jax.numpy as jnp
from jax.experimental import pallas as pl


def kernel(x, edge_index, W0, b0, gamma0, beta0, W1, b1, gamma1, beta1, W2, b2, gamma2, beta2, fc_W, fc_b):
    raise NotImplementedError("write your pallas kernel here")


# speedup vs baseline: 10.9386x; 10.9386x over previous
import functools

import jax
import jax.numpy as jnp
from jax import lax
from jax.experimental import pallas as pl
from jax.experimental.pallas import tpu as pltpu
from jax.experimental.pallas import tpu_sc as plsc

N = 10000
E = 320000
D = 128

NC = 2
NS = 16
NW = NC * NS
EPW = E // NW
CHUNK = 80
NCHUNK = EPW // CHUNK
ROWS_PT = 624
TAIL_BASE = NS * ROWS_PT
TAIL = N - TAIL_BASE
NP = 10240
DEG_PT = NP // NS

_MESH = plsc.VectorSubcoreMesh(core_axis_name="c", subcore_axis_name="s",
                               num_cores=NC, num_subcores=NS)


def _sc_deg_body(dst_hbm, out_hbm, deg_sh, ones_v, zeros_v, idx_v):
    cid = lax.axis_index("c")
    sid = lax.axis_index("s")
    wid = cid * NS + sid

    ones16 = jnp.ones((16,), jnp.float32)
    zeros16 = jnp.zeros((16,), jnp.float32)
    for i in range(CHUNK // 16):
        ones_v[pl.ds(16 * i, 16)] = ones16
    for i in range(DEG_PT // 16):
        zeros_v[pl.ds(16 * i, 16)] = zeros16

    pltpu.sync_copy(zeros_v, deg_sh.at[pl.ds(sid * DEG_PT, DEG_PT)])
    plsc.subcore_barrier()

    ebase = wid * EPW

    def step(i, carry):
        off = ebase + i * CHUNK
        pltpu.sync_copy(dst_hbm.at[pl.ds(off, CHUNK)], idx_v)
        pltpu.sync_copy(ones_v, deg_sh.at[idx_v], add=True)
        return carry

    lax.fori_loop(0, NCHUNK, step, 0)
    plsc.subcore_barrier()

    pltpu.sync_copy(
        deg_sh.at[pl.ds(sid * DEG_PT, DEG_PT)],
        out_hbm.at[pl.ds(cid * NP + sid * DEG_PT, DEG_PT)],
    )


_sc_deg = functools.partial(
    pl.kernel,
    out_type=jax.ShapeDtypeStruct((2 * NP,), jnp.float32),
    mesh=_MESH,
    scratch_types=[
        pltpu.VMEM_SHARED((NP,), jnp.float32),
        pltpu.VMEM((CHUNK,), jnp.float32),
        pltpu.VMEM((DEG_PT,), jnp.float32),
        pltpu.VMEM((CHUNK,), jnp.int32),
    ],
)(_sc_deg_body)


def _sc_agg_body(g_hbm, src_hbm, dst_hbm, out_hbm, acc_sh, sidx_v, didx_v,
                 rows_v, gsem):
    cid = lax.axis_index("c")
    sid = lax.axis_index("s")
    wid = cid * NS + sid

    rbase = sid * ROWS_PT
    pltpu.sync_copy(g_hbm.at[pl.ds(rbase, ROWS_PT)],
                    acc_sh.at[pl.ds(rbase, ROWS_PT)])

    @pl.when(sid == 0)
    def _():
        pltpu.sync_copy(g_hbm.at[pl.ds(TAIL_BASE, TAIL)],
                        acc_sh.at[pl.ds(TAIL_BASE, TAIL)])

    plsc.subcore_barrier()

    ebase = wid * EPW

    def step(i, carry):
        off = ebase + i * CHUNK
        pltpu.sync_copy(src_hbm.at[pl.ds(off, CHUNK)], sidx_v)
        pltpu.sync_copy(dst_hbm.at[pl.ds(off, CHUNK)], didx_v)
        pltpu.async_copy(g_hbm.at[sidx_v], rows_v, gsem).wait()
        pltpu.sync_copy(rows_v, acc_sh.at[didx_v], add=True)
        return carry

    lax.fori_loop(0, NCHUNK, step, 0)
    plsc.subcore_barrier()

    pltpu.sync_copy(acc_sh.at[pl.ds(rbase, ROWS_PT)],
                    out_hbm.at[pl.ds(cid * N + rbase, ROWS_PT)])

    @pl.when(sid == 0)
    def _():
        pltpu.sync_copy(acc_sh.at[pl.ds(TAIL_BASE, TAIL)],
                        out_hbm.at[pl.ds(cid * N + TAIL_BASE, TAIL)])


_sc_agg = functools.partial(
    pl.kernel,
    out_type=jax.ShapeDtypeStruct((2 * N, D), jnp.float32),
    mesh=_MESH,
    scratch_types=[
        pltpu.VMEM_SHARED((N, D), jnp.float32),
        pltpu.VMEM((CHUNK,), jnp.int32),
        pltpu.VMEM((CHUNK,), jnp.int32),
        pltpu.VMEM((CHUNK, D), jnp.float32),
        pltpu.SemaphoreType.DMA,
    ],
)(_sc_agg_body)


def _tc_pre_body(deg0, deg1, x, w0, g_out, dinv_out):
    dinv = lax.rsqrt(deg0[...] + deg1[...] + 1.0)
    dinv_out[...] = dinv
    g_out[...] = jnp.dot(x[...], w0[...],
                         preferred_element_type=jnp.float32) * dinv


_tc_pre = pl.pallas_call(
    _tc_pre_body,
    out_shape=[
        jax.ShapeDtypeStruct((N, D), jnp.float32),
        jax.ShapeDtypeStruct((N, 1), jnp.float32),
    ],
)


def _bn_relu(c, gamma, beta):
    m = jnp.mean(c, axis=0, keepdims=True)
    v = jnp.mean((c - m) * (c - m), axis=0, keepdims=True)
    return jax.nn.relu((c - m) * lax.rsqrt(v + 1e-5) * gamma + beta)


def _tc_mid_body(acc0, acc1, g, hprev, dinv, b, gamma, beta, w_next,
                 h_out, g_out, *, residual):
    c = dinv[...] * (acc0[...] + acc1[...] - g[...]) + b[...]
    if residual:
        c = hprev[...] + c
    h = _bn_relu(c, gamma[...], beta[...])
    h_out[...] = h
    g_out[...] = jnp.dot(h, w_next[...],
                         preferred_element_type=jnp.float32) * dinv[...]


def _tc_mid(residual):
    return pl.pallas_call(
        functools.partial(_tc_mid_body, residual=residual),
        out_shape=[
            jax.ShapeDtypeStruct((N, D), jnp.float32),
            jax.ShapeDtypeStruct((N, D), jnp.float32),
        ],
    )


_tc_mid0 = _tc_mid(False)
_tc_mid1 = _tc_mid(True)


def _tc_post_body(acc0, acc1, g, hprev, dinv, b, gamma, beta, fc_w, fc_b,
                  out):
    c = hprev[...] + dinv[...] * (acc0[...] + acc1[...] - g[...]) + b[...]
    h = _bn_relu(c, gamma[...], beta[...])
    logits = jnp.dot(h, fc_w[...], preferred_element_type=jnp.float32)
    out[...] = jax.nn.sigmoid(logits + fc_b[...])


_tc_post = pl.pallas_call(
    _tc_post_body,
    out_shape=jax.ShapeDtypeStruct((N, 1), jnp.float32),
)


def kernel(x, edge_index, W0, b0, gamma0, beta0, W1, b1, gamma1, beta1,
           W2, b2, gamma2, beta2, fc_W, fc_b):
    src = edge_index[0]
    dst = edge_index[1]

    degs = _sc_deg(dst)
    deg0 = degs[:N, None]
    deg1 = degs[NP:NP + N, None]

    g0, dinv = _tc_pre(deg0, deg1, x, W0)

    acc = _sc_agg(g0, src, dst)
    h1, g1 = _tc_mid0(acc[:N], acc[N:], g0, g0, dinv,
                      b0[None, :], gamma0[None, :], beta0[None, :], W1)

    acc = _sc_agg(g1, src, dst)
    h2, g2 = _tc_mid1(acc[:N], acc[N:], g1, h1, dinv,
                      b1[None, :], gamma1[None, :], beta1[None, :], W2)

    acc = _sc_agg(g2, src, dst)
    out = _tc_post(acc[:N], acc[N:], g2, h2, dinv,
                   b2[None, :], gamma2[None, :], beta2[None, :],
                   fc_W, fc_b[None, :])
    return out


# speedup vs baseline: 20.0448x; 1.8325x over previous
import functools

import jax
import jax.numpy as jnp
from jax import lax
from jax.experimental import pallas as pl
from jax.experimental.pallas import tpu as pltpu
from jax.experimental.pallas import tpu_sc as plsc

N = 10000
E = 320000
D = 128

NC = 2
NS = 16
NW = NC * NS
EPW = E // NW
CHUNK = 80
NCHUNK = EPW // CHUNK
NBUF = 2
ROWS_PT = 624
TAIL_BASE = NS * ROWS_PT
TAIL = N - TAIL_BASE
NP = 10240
DEG_PT = NP // NS

_MESH = plsc.VectorSubcoreMesh(core_axis_name="c", subcore_axis_name="s",
                               num_cores=NC, num_subcores=NS)


def _sc_deg_body(dst_hbm, out_hbm, deg_sh, ones_v, zeros_v, didx_v):
    cid = lax.axis_index("c")
    sid = lax.axis_index("s")
    wid = cid * NS + sid

    ones16 = jnp.ones((16,), jnp.float32)
    zeros16 = jnp.zeros((16,), jnp.float32)
    for i in range(CHUNK // 16):
        ones_v[pl.ds(16 * i, 16)] = ones16
    for i in range(DEG_PT // 16):
        zeros_v[pl.ds(16 * i, 16)] = zeros16

    pltpu.sync_copy(dst_hbm.at[wid], didx_v)
    pltpu.sync_copy(zeros_v, deg_sh.at[pl.ds(sid * DEG_PT, DEG_PT)])
    plsc.subcore_barrier()

    def step(i, carry):
        pltpu.sync_copy(ones_v, deg_sh.at[didx_v.at[i]], add=True)
        return carry

    lax.fori_loop(0, NCHUNK, step, 0)
    plsc.subcore_barrier()

    pltpu.sync_copy(
        deg_sh.at[pl.ds(sid * DEG_PT, DEG_PT)],
        out_hbm.at[pl.ds(cid * NP + sid * DEG_PT, DEG_PT)],
    )


_sc_deg = functools.partial(
    pl.kernel,
    out_type=jax.ShapeDtypeStruct((2 * NP,), jnp.float32),
    mesh=_MESH,
    scratch_types=[
        pltpu.VMEM_SHARED((NP,), jnp.float32),
        pltpu.VMEM((CHUNK,), jnp.float32),
        pltpu.VMEM((DEG_PT,), jnp.float32),
        pltpu.VMEM((NCHUNK, CHUNK), jnp.int32),
    ],
)(_sc_deg_body)


def _sc_agg_body(g_hbm, src_hbm, dst_hbm, out_hbm, acc_sh, sidx_v, didx_v,
                 *rows_and_sems):
    rows = rows_and_sems[:NBUF]
    sems = rows_and_sems[NBUF:]
    cid = lax.axis_index("c")
    sid = lax.axis_index("s")
    wid = cid * NS + sid

    pltpu.sync_copy(src_hbm.at[wid], sidx_v)
    pltpu.sync_copy(dst_hbm.at[wid], didx_v)

    rbase = sid * ROWS_PT
    pltpu.sync_copy(g_hbm.at[pl.ds(rbase, ROWS_PT)],
                    acc_sh.at[pl.ds(rbase, ROWS_PT)])

    @pl.when(sid == 0)
    def _():
        pltpu.sync_copy(g_hbm.at[pl.ds(TAIL_BASE, TAIL)],
                        acc_sh.at[pl.ds(TAIL_BASE, TAIL)])

    plsc.subcore_barrier()

    pltpu.async_copy(g_hbm.at[sidx_v.at[pl.ds(0, CHUNK)]], rows[0], sems[0])

    def outer(k, carry):
        i0 = k * NBUF
        for b in range(NBUF):
            i = i0 + b

            @pl.when(i < NCHUNK)
            def _():
                pltpu.make_async_copy(g_hbm.at[pl.ds(0, CHUNK)],
                                      rows[b], sems[b]).wait()
                nxt = i + 1
                nb = (b + 1) % NBUF

                @pl.when(nxt < NCHUNK)
                def _():
                    pltpu.async_copy(
                        g_hbm.at[sidx_v.at[pl.ds(nxt * CHUNK, CHUNK)]],
                        rows[nb], sems[nb])

                pltpu.sync_copy(rows[b], acc_sh.at[didx_v.at[i]], add=True)
        return carry

    lax.fori_loop(0, (NCHUNK + NBUF - 1) // NBUF, outer, 0)
    plsc.subcore_barrier()

    pltpu.sync_copy(acc_sh.at[pl.ds(rbase, ROWS_PT)],
                    out_hbm.at[pl.ds(cid * N + rbase, ROWS_PT)])

    @pl.when(sid == 0)
    def _():
        pltpu.sync_copy(acc_sh.at[pl.ds(TAIL_BASE, TAIL)],
                        out_hbm.at[pl.ds(cid * N + TAIL_BASE, TAIL)])


_sc_agg = functools.partial(
    pl.kernel,
    out_type=jax.ShapeDtypeStruct((2 * N, D), jnp.float32),
    mesh=_MESH,
    scratch_types=(
        [pltpu.VMEM_SHARED((N, D), jnp.float32),
         pltpu.VMEM((EPW,), jnp.int32),
         pltpu.VMEM((NCHUNK, CHUNK), jnp.int32)]
        + [pltpu.VMEM((CHUNK, D), jnp.float32)] * NBUF
        + [pltpu.SemaphoreType.DMA] * NBUF
    ),
)(_sc_agg_body)


def _tc_pre_body(deg0, deg1, x, w0, g_out, dinv_out):
    dinv = lax.rsqrt(deg0[...] + deg1[...] + 1.0)
    dinv_out[...] = dinv
    g_out[...] = jnp.dot(x[...], w0[...],
                         preferred_element_type=jnp.float32) * dinv


_tc_pre = pl.pallas_call(
    _tc_pre_body,
    out_shape=[
        jax.ShapeDtypeStruct((N, D), jnp.float32),
        jax.ShapeDtypeStruct((N, 1), jnp.float32),
    ],
)


def _bn_relu(c, gamma, beta):
    m = jnp.mean(c, axis=0, keepdims=True)
    v = jnp.mean((c - m) * (c - m), axis=0, keepdims=True)
    return jax.nn.relu((c - m) * lax.rsqrt(v + 1e-5) * gamma + beta)


def _tc_mid_body(acc0, acc1, g, hprev, dinv, b, gamma, beta, w_next,
                 h_out, g_out, *, residual):
    c = dinv[...] * (acc0[...] + acc1[...] - g[...]) + b[...]
    if residual:
        c = hprev[...] + c
    h = _bn_relu(c, gamma[...], beta[...])
    h_out[...] = h
    g_out[...] = jnp.dot(h, w_next[...],
                         preferred_element_type=jnp.float32) * dinv[...]


def _tc_mid(residual):
    return pl.pallas_call(
        functools.partial(_tc_mid_body, residual=residual),
        out_shape=[
            jax.ShapeDtypeStruct((N, D), jnp.float32),
            jax.ShapeDtypeStruct((N, D), jnp.float32),
        ],
    )


_tc_mid0 = _tc_mid(False)
_tc_mid1 = _tc_mid(True)


def _tc_post_body(acc0, acc1, g, hprev, dinv, b, gamma, beta, fc_w, fc_b,
                  out):
    c = hprev[...] + dinv[...] * (acc0[...] + acc1[...] - g[...]) + b[...]
    h = _bn_relu(c, gamma[...], beta[...])
    logits = jnp.dot(h, fc_w[...], preferred_element_type=jnp.float32)
    out[...] = jax.nn.sigmoid(logits + fc_b[...])


_tc_post = pl.pallas_call(
    _tc_post_body,
    out_shape=jax.ShapeDtypeStruct((N, 1), jnp.float32),
)


def kernel(x, edge_index, W0, b0, gamma0, beta0, W1, b1, gamma1, beta1,
           W2, b2, gamma2, beta2, fc_W, fc_b):
    src2 = edge_index[0].reshape(NW, EPW)
    dst3 = edge_index[1].reshape(NW, NCHUNK, CHUNK)

    degs = _sc_deg(dst3)
    deg0 = degs[:N, None]
    deg1 = degs[NP:NP + N, None]

    g0, dinv = _tc_pre(deg0, deg1, x, W0)

    acc = _sc_agg(g0, src2, dst3)
    h1, g1 = _tc_mid0(acc[:N], acc[N:], g0, g0, dinv,
                      b0[None, :], gamma0[None, :], beta0[None, :], W1)

    acc = _sc_agg(g1, src2, dst3)
    h2, g2 = _tc_mid1(acc[:N], acc[N:], g1, h1, dinv,
                      b1[None, :], gamma1[None, :], beta1[None, :], W2)

    acc = _sc_agg(g2, src2, dst3)
    out = _tc_post(acc[:N], acc[N:], g2, h2, dinv,
                   b2[None, :], gamma2[None, :], beta2[None, :],
                   fc_W, fc_b[None, :])
    return out


# speedup vs baseline: 24.5727x; 1.2259x over previous
import functools

import jax
import jax.numpy as jnp
from jax import lax
from jax.experimental import pallas as pl
from jax.experimental.pallas import tpu as pltpu
from jax.experimental.pallas import tpu_sc as plsc

N = 10000
E = 320000
D = 128

NC = 2
NS = 16
NW = NC * NS
EPW = E // NW
CHUNK = 80
NCHUNK = EPW // CHUNK
NBUF = 2
ROWS_PT = 624
TAIL_BASE = NS * ROWS_PT
TAIL = N - TAIL_BASE
NP = 10240
DEG_PT = NP // NS

_MESH = plsc.VectorSubcoreMesh(core_axis_name="c", subcore_axis_name="s",
                               num_cores=NC, num_subcores=NS)


def _sc_deg_body(dst_hbm, out_hbm, deg_sh, ones_v, zeros_v, didx_v):
    cid = lax.axis_index("c")
    sid = lax.axis_index("s")
    wid = cid * NS + sid

    ones16 = jnp.ones((16,), jnp.float32)
    zeros16 = jnp.zeros((16,), jnp.float32)
    for i in range(CHUNK // 16):
        ones_v[pl.ds(16 * i, 16)] = ones16
    for i in range(DEG_PT // 16):
        zeros_v[pl.ds(16 * i, 16)] = zeros16

    pltpu.sync_copy(dst_hbm.at[wid], didx_v)
    pltpu.sync_copy(zeros_v, deg_sh.at[pl.ds(sid * DEG_PT, DEG_PT)])
    plsc.subcore_barrier()

    def step(i, carry):
        pltpu.sync_copy(ones_v, deg_sh.at[didx_v.at[i]], add=True)
        return carry

    lax.fori_loop(0, NCHUNK, step, 0)
    plsc.subcore_barrier()

    pltpu.sync_copy(
        deg_sh.at[pl.ds(sid * DEG_PT, DEG_PT)],
        out_hbm.at[pl.ds(cid * NP + sid * DEG_PT, DEG_PT)],
    )


_sc_deg = functools.partial(
    pl.kernel,
    out_type=jax.ShapeDtypeStruct((2 * NP,), jnp.float32),
    mesh=_MESH,
    scratch_types=[
        pltpu.VMEM_SHARED((NP,), jnp.float32),
        pltpu.VMEM((CHUNK,), jnp.float32),
        pltpu.VMEM((DEG_PT,), jnp.float32),
        pltpu.VMEM((NCHUNK, CHUNK), jnp.int32),
    ],
)(_sc_deg_body)


def _sc_agg_body(g_hbm, src_hbm, dst_hbm, out_hbm, acc_sh, sidx_v, didx_v,
                 *rows_and_sems):
    rows = rows_and_sems[:NBUF]
    sems = rows_and_sems[NBUF:]
    cid = lax.axis_index("c")
    sid = lax.axis_index("s")
    wid = cid * NS + sid

    pltpu.sync_copy(src_hbm.at[wid], sidx_v)
    pltpu.sync_copy(dst_hbm.at[wid], didx_v)

    rbase = sid * ROWS_PT
    pltpu.sync_copy(g_hbm.at[pl.ds(rbase, ROWS_PT)],
                    acc_sh.at[pl.ds(rbase, ROWS_PT)])

    @pl.when(sid == 0)
    def _():
        pltpu.sync_copy(g_hbm.at[pl.ds(TAIL_BASE, TAIL)],
                        acc_sh.at[pl.ds(TAIL_BASE, TAIL)])

    plsc.subcore_barrier()

    gsems = sems[:NBUF]
    ssems = sems[NBUF:]

    def _drain(buf, sem):
        pltpu.make_async_copy(g_hbm.at[pl.ds(0, CHUNK)], buf, sem).wait()

    pltpu.async_copy(g_hbm.at[sidx_v.at[pl.ds(0, CHUNK)]], rows[0], gsems[0])

    def outer(k, carry):
        i0 = k * NBUF
        for b in range(NBUF):
            i = i0 + b
            nb = (b + 1) % NBUF

            @pl.when(i + 1 < NCHUNK)
            def _():
                @pl.when(i >= 1)
                def _():
                    _drain(rows[nb], ssems[nb])

                pltpu.async_copy(
                    g_hbm.at[sidx_v.at[pl.ds((i + 1) * CHUNK, CHUNK)]],
                    rows[nb], gsems[nb])

            @pl.when(i < NCHUNK)
            def _():
                _drain(rows[b], gsems[b])
                pltpu.async_copy(rows[b], acc_sh.at[didx_v.at[i]], ssems[b],
                                 add=True)
        return carry

    lax.fori_loop(0, (NCHUNK + NBUF - 1) // NBUF, outer, 0)
    _drain(rows[(NCHUNK - 2) % NBUF], ssems[(NCHUNK - 2) % NBUF])
    _drain(rows[(NCHUNK - 1) % NBUF], ssems[(NCHUNK - 1) % NBUF])
    plsc.subcore_barrier()

    pltpu.sync_copy(acc_sh.at[pl.ds(rbase, ROWS_PT)],
                    out_hbm.at[pl.ds(cid * N + rbase, ROWS_PT)])

    @pl.when(sid == 0)
    def _():
        pltpu.sync_copy(acc_sh.at[pl.ds(TAIL_BASE, TAIL)],
                        out_hbm.at[pl.ds(cid * N + TAIL_BASE, TAIL)])


_sc_agg = functools.partial(
    pl.kernel,
    out_type=jax.ShapeDtypeStruct((2 * N, D), jnp.float32),
    mesh=_MESH,
    scratch_types=(
        [pltpu.VMEM_SHARED((N, D), jnp.float32),
         pltpu.VMEM((EPW,), jnp.int32),
         pltpu.VMEM((NCHUNK, CHUNK), jnp.int32)]
        + [pltpu.VMEM((CHUNK, D), jnp.float32)] * NBUF
        + [pltpu.SemaphoreType.DMA] * (2 * NBUF)
    ),
)(_sc_agg_body)


def _tc_pre_body(deg0, deg1, x, w0, g_out, dinv_out):
    dinv = lax.rsqrt(deg0[...] + deg1[...] + 1.0)
    dinv_out[...] = dinv
    g_out[...] = jnp.dot(x[...], w0[...],
                         preferred_element_type=jnp.float32) * dinv


_tc_pre = pl.pallas_call(
    _tc_pre_body,
    out_shape=[
        jax.ShapeDtypeStruct((N, D), jnp.float32),
        jax.ShapeDtypeStruct((N, 1), jnp.float32),
    ],
)


def _bn_relu(c, gamma, beta):
    m = jnp.mean(c, axis=0, keepdims=True)
    v = jnp.mean((c - m) * (c - m), axis=0, keepdims=True)
    return jax.nn.relu((c - m) * lax.rsqrt(v + 1e-5) * gamma + beta)


def _tc_mid_body(acc0, acc1, g, hprev, dinv, b, gamma, beta, w_next,
                 h_out, g_out, *, residual):
    c = dinv[...] * (acc0[...] + acc1[...] - g[...]) + b[...]
    if residual:
        c = hprev[...] + c
    h = _bn_relu(c, gamma[...], beta[...])
    h_out[...] = h
    g_out[...] = jnp.dot(h, w_next[...],
                         preferred_element_type=jnp.float32) * dinv[...]


def _tc_mid(residual):
    return pl.pallas_call(
        functools.partial(_tc_mid_body, residual=residual),
        out_shape=[
            jax.ShapeDtypeStruct((N, D), jnp.float32),
            jax.ShapeDtypeStruct((N, D), jnp.float32),
        ],
    )


_tc_mid0 = _tc_mid(False)
_tc_mid1 = _tc_mid(True)


def _tc_post_body(acc0, acc1, g, hprev, dinv, b, gamma, beta, fc_w, fc_b,
                  out):
    c = hprev[...] + dinv[...] * (acc0[...] + acc1[...] - g[...]) + b[...]
    h = _bn_relu(c, gamma[...], beta[...])
    logits = jnp.dot(h, fc_w[...], preferred_element_type=jnp.float32)
    out[...] = jax.nn.sigmoid(logits + fc_b[...])


_tc_post = pl.pallas_call(
    _tc_post_body,
    out_shape=jax.ShapeDtypeStruct((N, 1), jnp.float32),
)


def kernel(x, edge_index, W0, b0, gamma0, beta0, W1, b1, gamma1, beta1,
           W2, b2, gamma2, beta2, fc_W, fc_b):
    src2 = edge_index[0].reshape(NW, EPW)
    dst3 = edge_index[1].reshape(NW, NCHUNK, CHUNK)

    degs = _sc_deg(dst3)
    deg0 = degs[:N, None]
    deg1 = degs[NP:NP + N, None]

    g0, dinv = _tc_pre(deg0, deg1, x, W0)

    acc = _sc_agg(g0, src2, dst3)
    h1, g1 = _tc_mid0(acc[:N], acc[N:], g0, g0, dinv,
                      b0[None, :], gamma0[None, :], beta0[None, :], W1)

    acc = _sc_agg(g1, src2, dst3)
    h2, g2 = _tc_mid1(acc[:N], acc[N:], g1, h1, dinv,
                      b1[None, :], gamma1[None, :], beta1[None, :], W2)

    acc = _sc_agg(g2, src2, dst3)
    out = _tc_post(acc[:N], acc[N:], g2, h2, dinv,
                   b2[None, :], gamma2[None, :], beta2[None, :],
                   fc_W, fc_b[None, :])
    return out
